# baseline (speedup 1.0000x reference)
import functools

import jax
import jax.numpy as jnp
from jax import lax
from jax.experimental import pallas as pl
from jax.experimental.pallas import tpu as pltpu
from jax.experimental.pallas import tpu_sc as plsc

N = 10000
E = 320000
F_IN = 128
F_OUT = 64
R = 2
NB = 30

NC = 2
NS = 16
NW = NC * NS
CHUNK = 64
CPT = 160
CPG = 4
NGRP = CPT // CPG
EPT = CPT * CHUNK
E_PAD = NW * EPT
ROWS2D = E_PAD // CHUNK
NBUF = 2
ACC_ROWS = 10112
ZROWS = ACC_ROWS // NS
TBL_PAD = 20096
TROWS = TBL_PAD // NS


def _mm_body(att_ref, x_ref, basis_ref, arr_ref, h_ref, gidx_ref):
    gidx_ref[...] = arr_ref[2] * N + arr_ref[0]
    x = x_ref[...]
    for r in range(R):
        w_r = att_ref[r, 0] * basis_ref[0]
        for b in range(1, NB):
            w_r = w_r + att_ref[r, b] * basis_ref[b]
        h_ref[r * N:(r + 1) * N, :] = jnp.dot(
            x, w_r, preferred_element_type=jnp.float32)


def _build_table(att, x, basis, arr):
    return pl.pallas_call(
        _mm_body,
        out_shape=(jax.ShapeDtypeStruct((TBL_PAD, F_OUT), jnp.float32),
                   jax.ShapeDtypeStruct((ROWS2D, CHUNK), jnp.int32)),
        in_specs=[
            pl.BlockSpec(memory_space=pltpu.SMEM),
            pl.BlockSpec(memory_space=pltpu.VMEM),
            pl.BlockSpec(memory_space=pltpu.VMEM),
            pl.BlockSpec(memory_space=pltpu.VMEM),
        ],
    )(att, x, basis, arr)


def _edge_body(gidx_hbm, arr_hbm, h_hbm, zeros_hbm, out_hbm,
               gidx_g, dst_g, rows_v, table_sp, acc, gsems, asems, isems,
               ssem):
    c = lax.axis_index("c")
    s = lax.axis_index("s")
    w = s * NC + c
    base = w * CPT

    def load_slab(h, slot):
        pltpu.async_copy(gidx_hbm.at[pl.ds(base + h * CPG, CPG)],
                         gidx_g.at[slot], isems.at[slot])
        pltpu.async_copy(arr_hbm.at[1, pl.ds(base + h * CPG, CPG)],
                         dst_g.at[slot], isems.at[slot])

    def wait_slab(h, slot):
        pltpu.make_async_copy(gidx_hbm.at[pl.ds(base + h * CPG, CPG)],
                              gidx_g.at[slot], isems.at[slot]).wait()
        pltpu.make_async_copy(arr_hbm.at[1, pl.ds(base + h * CPG, CPG)],
                              dst_g.at[slot], isems.at[slot]).wait()

    load_slab(0, 0)
    load_slab(1, 1)

    pltpu.async_copy(h_hbm.at[pl.ds(s * TROWS, TROWS)],
                     table_sp.at[pl.ds(s * TROWS, TROWS)], ssem)
    pltpu.async_copy(zeros_hbm, acc.at[pl.ds(s * ZROWS, ZROWS)], ssem)
    pltpu.make_async_copy(h_hbm.at[pl.ds(s * TROWS, TROWS)],
                          table_sp.at[pl.ds(s * TROWS, TROWS)], ssem).wait()
    pltpu.make_async_copy(zeros_hbm, acc.at[pl.ds(s * ZROWS, ZROWS)],
                          ssem).wait()

    plsc.subcore_barrier()

    def wait_gather(t, slot, k, b):
        pltpu.make_async_copy(table_sp.at[gidx_g.at[slot, k]],
                              rows_v.at[b], gsems.at[b]).wait()

    def start_gather(t, slot, k, b):
        pltpu.async_copy(table_sp.at[gidx_g.at[slot, k]],
                         rows_v.at[b], gsems.at[b])

    def wait_scatter(t, slot, k, b):
        pltpu.make_async_copy(rows_v.at[b], acc.at[dst_g.at[slot, k]],
                              asems.at[b]).wait()

    def start_scatter(t, slot, k, b):
        pltpu.async_copy(rows_v.at[b], acc.at[dst_g.at[slot, k]],
                         asems.at[b], add=True)

    def sk(t):
        return ((t // CPG) % 2, t % CPG, t % NBUF)

    def turn(t, ph, load_h=None, load_slot=None, wait_h=None,
             wait_slot=None, first=False, last=False):
        slot, k, b = sk(ph)
        pslot, pk, pb = sk(ph - 1) if ph >= 1 else (0, 0, 0)
        nslot, nk, nb = sk(ph + 1)
        if not first:
            wait_scatter(t - 1, pslot, pk, pb)
        if load_h is not None:
            load_slab(load_h, load_slot)
        if wait_h is not None:
            wait_slab(wait_h, wait_slot)
        if not last:
            start_gather(t + 1, nslot, nk, nb)
        wait_gather(t, slot, k, b)
        start_scatter(t, slot, k, b)

    wait_slab(0, 0)
    start_gather(0, 0, 0, 0)
    turn(0, 0, first=True)
    turn(1, 1)
    turn(2, 2)
    turn(3, 3, wait_h=1, wait_slot=1)

    def pair(p, carry):
        t0 = 8 * p + 4
        for half in range(2):
            h = 2 * p + 1 + half
            for k in range(CPG):
                t = t0 + 4 * half + k
                ph = 4 * (1 + half) + k
                turn(t, ph,
                     load_h=(h + 1) if k == 0 else None,
                     load_slot=half if k == 0 else None,
                     wait_h=(h + 1) if k == CPG - 1 else None,
                     wait_slot=half if k == CPG - 1 else None)
        return carry

    lax.fori_loop(0, (NGRP - 2) // 2, pair, 0)

    turn(156, 156)
    turn(157, 157)
    turn(158, 158)
    turn(159, 159, last=True)
    wait_scatter(159, *sk(159))

    plsc.subcore_barrier()
    pltpu.sync_copy(acc.at[pl.ds(s * ZROWS, ZROWS)],
                    out_hbm.at[c].at[pl.ds(s * ZROWS, ZROWS)])


@functools.cache
def _edge_call():
    return pl.kernel(
        _edge_body,
        out_type=jax.ShapeDtypeStruct((NC, ACC_ROWS, F_OUT), jnp.float32),
        mesh=plsc.VectorSubcoreMesh(core_axis_name="c", subcore_axis_name="s",
                                    num_cores=NC, num_subcores=NS),
        compiler_params=pltpu.CompilerParams(use_tc_tiling_on_sc=False),
        scratch_types=[
            pltpu.VMEM((2, CPG, CHUNK), jnp.int32),
            pltpu.VMEM((2, CPG, CHUNK), jnp.int32),
            pltpu.VMEM((NBUF, CHUNK, F_OUT), jnp.float32),
            pltpu.VMEM_SHARED((TBL_PAD, F_OUT), jnp.float32),
            pltpu.VMEM_SHARED((ACC_ROWS, F_OUT), jnp.float32),
            pltpu.SemaphoreType.DMA((NBUF,)),
            pltpu.SemaphoreType.DMA((NBUF,)),
            pltpu.SemaphoreType.DMA((2,)),
            pltpu.SemaphoreType.DMA,
        ],
    )


def _combine_body(p_ref, x_ref, root_ref, bias_ref, o_ref):
    o_ref[...] = (p_ref[0, :N] + p_ref[1, :N]
                  + jnp.dot(x_ref[...], root_ref[...],
                            preferred_element_type=jnp.float32)
                  + bias_ref[...])


def _combine(partials, x, root, bias2d):
    return pl.pallas_call(
        _combine_body,
        out_shape=jax.ShapeDtypeStruct((N, F_OUT), jnp.float32),
    )(partials, x, root, bias2d)


def kernel(x, edge_index, edge_type, basis, att, root, bias):
    pad = E_PAD - E
    pad_blk = jnp.stack([
        jnp.zeros((pad,), jnp.int32),
        N + (jnp.arange(pad, dtype=jnp.int32) % (ACC_ROWS - N)),
        jnp.zeros((pad,), jnp.int32)])
    arr = jnp.concatenate(
        [jnp.concatenate([edge_index, edge_type[None]], axis=0), pad_blk],
        axis=1).reshape(3, ROWS2D, CHUNK)
    zeros = jnp.zeros((ZROWS, F_OUT), jnp.float32)

    table, gidx2d = _build_table(att, x, basis, arr)
    partials = _edge_call()(gidx2d, arr, table, zeros)
    return _combine(partials, x, root, bias.reshape(1, F_OUT))


# speedup vs baseline: 1.0664x; 1.0664x over previous
import functools

import jax
import jax.numpy as jnp
from jax import lax
from jax.experimental import pallas as pl
from jax.experimental.pallas import tpu as pltpu
from jax.experimental.pallas import tpu_sc as plsc

N = 10000
E = 320000
F_IN = 128
F_OUT = 64
R = 2
NB = 30

NC = 2
NS = 16
NW = NC * NS
CHUNK = 64
CPT = 160
CPG = 4
NGRP = CPT // CPG
EPT = CPT * CHUNK
E_PAD = NW * EPT
ROWS2D = E_PAD // CHUNK
NBUF = 2
ACC_ROWS = 10112
ZROWS = ACC_ROWS // NS
TBL_PAD = 20096
TROWS = TBL_PAD // NS


def _mm_body(att_ref, x_ref, basis_ref, src_ref, typ_ref, h_ref, gidx_ref):
    gidx_ref[...] = typ_ref[...] * N + src_ref[...]
    x = x_ref[...]
    for r in range(R):
        w_r = att_ref[r, 0] * basis_ref[0]
        for b in range(1, NB):
            w_r = w_r + att_ref[r, b] * basis_ref[b]
        h_ref[r * N:(r + 1) * N, :] = jnp.dot(
            x, w_r, preferred_element_type=jnp.float32)


def _build_table(att, x, basis, src2d, typ2d):
    return pl.pallas_call(
        _mm_body,
        out_shape=(jax.ShapeDtypeStruct((TBL_PAD, F_OUT), jnp.float32),
                   jax.ShapeDtypeStruct((ROWS2D, CHUNK), jnp.int32)),
        in_specs=[
            pl.BlockSpec(memory_space=pltpu.SMEM),
            pl.BlockSpec(memory_space=pltpu.VMEM),
            pl.BlockSpec(memory_space=pltpu.VMEM),
            pl.BlockSpec(memory_space=pltpu.VMEM),
            pl.BlockSpec(memory_space=pltpu.VMEM),
        ],
    )(att, x, basis, src2d, typ2d)


def _edge_body(gidx_hbm, dst_hbm, h_hbm, zeros_hbm, out_hbm,
               gidx_g, dst_g, rows_v, table_sp, acc, gsems, asems, isems,
               ssem):
    c = lax.axis_index("c")
    s = lax.axis_index("s")
    w = s * NC + c
    base = w * CPT

    def load_slab(h, slot):
        pltpu.async_copy(gidx_hbm.at[pl.ds(base + h * CPG, CPG)],
                         gidx_g.at[slot], isems.at[slot])
        pltpu.async_copy(dst_hbm.at[pl.ds(base + h * CPG, CPG)],
                         dst_g.at[slot], isems.at[slot])

    def wait_slab(h, slot):
        pltpu.make_async_copy(gidx_hbm.at[pl.ds(base + h * CPG, CPG)],
                              gidx_g.at[slot], isems.at[slot]).wait()
        pltpu.make_async_copy(dst_hbm.at[pl.ds(base + h * CPG, CPG)],
                              dst_g.at[slot], isems.at[slot]).wait()

    load_slab(0, 0)
    load_slab(1, 1)

    pltpu.async_copy(h_hbm.at[pl.ds(s * TROWS, TROWS)],
                     table_sp.at[pl.ds(s * TROWS, TROWS)], ssem)
    pltpu.async_copy(zeros_hbm, acc.at[pl.ds(s * ZROWS, ZROWS)], ssem)
    pltpu.make_async_copy(h_hbm.at[pl.ds(s * TROWS, TROWS)],
                          table_sp.at[pl.ds(s * TROWS, TROWS)], ssem).wait()
    pltpu.make_async_copy(zeros_hbm, acc.at[pl.ds(s * ZROWS, ZROWS)],
                          ssem).wait()

    plsc.subcore_barrier()

    def wait_gather(t, slot, k, b):
        pltpu.make_async_copy(table_sp.at[gidx_g.at[slot, k]],
                              rows_v.at[b], gsems.at[b]).wait()

    def start_gather(t, slot, k, b):
        pltpu.async_copy(table_sp.at[gidx_g.at[slot, k]],
                         rows_v.at[b], gsems.at[b])

    def wait_scatter(t, slot, k, b):
        pltpu.make_async_copy(rows_v.at[b], acc.at[dst_g.at[slot, k]],
                              asems.at[b]).wait()

    def start_scatter(t, slot, k, b):
        pltpu.async_copy(rows_v.at[b], acc.at[dst_g.at[slot, k]],
                         asems.at[b], add=True)

    def sk(t):
        return ((t // CPG) % 2, t % CPG, t % NBUF)

    def turn(t, ph, load_h=None, load_slot=None, wait_h=None,
             wait_slot=None, first=False, last=False):
        slot, k, b = sk(ph)
        pslot, pk, pb = sk(ph - 1) if ph >= 1 else (0, 0, 0)
        nslot, nk, nb = sk(ph + 1)
        if not first:
            wait_scatter(t - 1, pslot, pk, pb)
        if load_h is not None:
            load_slab(load_h, load_slot)
        if wait_h is not None:
            wait_slab(wait_h, wait_slot)
        if not last:
            start_gather(t + 1, nslot, nk, nb)
        wait_gather(t, slot, k, b)
        start_scatter(t, slot, k, b)

    wait_slab(0, 0)
    start_gather(0, 0, 0, 0)
    turn(0, 0, first=True)
    turn(1, 1)
    turn(2, 2)
    turn(3, 3, wait_h=1, wait_slot=1)

    def pair(p, carry):
        t0 = 8 * p + 4
        for half in range(2):
            h = 2 * p + 1 + half
            for k in range(CPG):
                t = t0 + 4 * half + k
                ph = 4 * (1 + half) + k
                turn(t, ph,
                     load_h=(h + 1) if k == 0 else None,
                     load_slot=half if k == 0 else None,
                     wait_h=(h + 1) if k == CPG - 1 else None,
                     wait_slot=half if k == CPG - 1 else None)
        return carry

    lax.fori_loop(0, (NGRP - 2) // 2, pair, 0)

    turn(156, 156)
    turn(157, 157)
    turn(158, 158)
    turn(159, 159, last=True)
    wait_scatter(159, *sk(159))

    plsc.subcore_barrier()
    pltpu.sync_copy(acc.at[pl.ds(s * ZROWS, ZROWS)],
                    out_hbm.at[c].at[pl.ds(s * ZROWS, ZROWS)])


@functools.cache
def _edge_call():
    return pl.kernel(
        _edge_body,
        out_type=jax.ShapeDtypeStruct((NC, ACC_ROWS, F_OUT), jnp.float32),
        mesh=plsc.VectorSubcoreMesh(core_axis_name="c", subcore_axis_name="s",
                                    num_cores=NC, num_subcores=NS),
        compiler_params=pltpu.CompilerParams(use_tc_tiling_on_sc=False),
        scratch_types=[
            pltpu.VMEM((2, CPG, CHUNK), jnp.int32),
            pltpu.VMEM((2, CPG, CHUNK), jnp.int32),
            pltpu.VMEM((NBUF, CHUNK, F_OUT), jnp.float32),
            pltpu.VMEM_SHARED((TBL_PAD, F_OUT), jnp.float32),
            pltpu.VMEM_SHARED((ACC_ROWS, F_OUT), jnp.float32),
            pltpu.SemaphoreType.DMA((NBUF,)),
            pltpu.SemaphoreType.DMA((NBUF,)),
            pltpu.SemaphoreType.DMA((2,)),
            pltpu.SemaphoreType.DMA,
        ],
    )


def _combine_body(p_ref, x_ref, root_ref, bias_ref, o_ref):
    o_ref[...] = (p_ref[0, :N] + p_ref[1, :N]
                  + jnp.dot(x_ref[...], root_ref[...],
                            preferred_element_type=jnp.float32)
                  + bias_ref[...])


def _combine(partials, x, root, bias2d):
    return pl.pallas_call(
        _combine_body,
        out_shape=jax.ShapeDtypeStruct((N, F_OUT), jnp.float32),
    )(partials, x, root, bias2d)


def kernel(x, edge_index, edge_type, basis, att, root, bias):
    src = edge_index[0]
    dst = edge_index[1]
    pad = E_PAD - E
    src_p = jnp.concatenate([src, jnp.zeros((pad,), jnp.int32)])
    typ_p = jnp.concatenate([edge_type, jnp.zeros((pad,), jnp.int32)])
    dst_p = jnp.concatenate(
        [dst, N + (jnp.arange(pad, dtype=jnp.int32) % (ACC_ROWS - N))])
    src2d = src_p.reshape(ROWS2D, CHUNK)
    typ2d = typ_p.reshape(ROWS2D, CHUNK)
    dst2d = dst_p.reshape(ROWS2D, CHUNK)
    zeros = jnp.zeros((ZROWS, F_OUT), jnp.float32)

    table, gidx2d = _build_table(att, x, basis, src2d, typ2d)
    partials = _edge_call()(gidx2d, dst2d, table, zeros)
    return _combine(partials, x, root, bias.reshape(1, F_OUT))
